# Initial kernel scaffold; baseline (speedup 1.0000x reference)
import functools

import jax
import jax.numpy as jnp
from jax import lax
from jax.experimental import pallas as pl
from jax.experimental.pallas import tpu as pltpu
from jax.experimental.pallas import tpu_sc as plsc

N = 10000
E = 160000
SEQ = 12
B = 4
BN = B * N
TAU = 0.5
HID_GL = 256
HID_FC = 64
HORIZON = 12

NC, NS = 2, 16
NW = NC * NS
E_PAD = 163840
EPT = E_PAD // NW
NV = EPT // 16
CHUNK = 128
NCH = EPT // CHUNK
AGG_SP = 40960
TRASH = BN
XPAD = 16


def _embed_body(ei_ref, w1_ref, w2c_ref, gum_ref, p_ref, g_ref):
    z = jnp.maximum(
        jnp.dot(ei_ref[...], w1_ref[...], preferred_element_type=jnp.float32),
        0.0)
    p_ref[...] = jnp.dot(z, w2c_ref[...], preferred_element_type=jnp.float32)
    u = gum_ref[...]
    g_ref[...] = -jnp.log(-jnp.log(u + 1e-10) + 1e-10)


def _embed(entire_inputs, w1, w2cat, gum_t):
    blk_n = 1000
    blk_e = E_PAD // 10
    return pl.pallas_call(
        _embed_body,
        grid=(10,),
        in_specs=[
            pl.BlockSpec((blk_n, 1000), lambda i: (i, 0)),
            pl.BlockSpec((1000, HID_GL), lambda i: (0, 0)),
            pl.BlockSpec((HID_GL, 4), lambda i: (0, 0)),
            pl.BlockSpec((2, blk_e), lambda i: (0, i)),
        ],
        out_specs=[
            pl.BlockSpec((blk_n, 4), lambda i: (i, 0)),
            pl.BlockSpec((2, blk_e), lambda i: (0, i)),
        ],
        out_shape=[
            jax.ShapeDtypeStruct((N, 4), jnp.float32),
            jax.ShapeDtypeStruct((2, E_PAD), jnp.float32),
        ],
    )(entire_inputs, w1, w2cat, gum_t)


def _sc_body(p_hbm, src_hbm, dst_hbm, g0_hbm, g1_hbm, x_hbm,
             samp_hbm, agg_hbm,
             p_v, src_v, dst_v, g0_v, g1_v, samp_v,
             gi_v, rows_v, zero_v,
             agg_sh, sem):
    c = lax.axis_index("c")
    s = lax.axis_index("s")
    tile = c * NS + s
    ebase = tile * EPT

    pltpu.sync_copy(src_hbm.at[pl.ds(ebase, EPT)], src_v)
    pltpu.sync_copy(dst_hbm.at[pl.ds(ebase, EPT)], dst_v)
    pltpu.sync_copy(g0_hbm.at[pl.ds(ebase, EPT)], g0_v)
    pltpu.sync_copy(g1_hbm.at[pl.ds(ebase, EPT)], g1_v)
    pltpu.sync_copy(p_hbm, p_v)

    def _zinit(i, carry):
        zero_v[i] = jnp.zeros((16,), jnp.float32)
        return carry
    lax.fori_loop(0, 256, _zinit, 0)
    arows = AGG_SP // NS

    def _zcopy(j, carry):
        pltpu.sync_copy(zero_v, agg_sh.at[pl.ds(s * arows + j * 256, 256)])
        return carry
    lax.fori_loop(0, arows // 256, _zcopy, 0)

    inv_tau = 1.0 / TAU

    def _sample(i, carry):
        sl = pl.ds(i * 16, 16)
        sv = src_v[sl]
        dv = dst_v[sl]
        s4 = sv * 4
        d4 = dv * 4
        ps0 = plsc.load_gather(p_v, [s4])
        ps1 = plsc.load_gather(p_v, [s4 + 1])
        pd0 = plsc.load_gather(p_v, [d4 + 2])
        pd1 = plsc.load_gather(p_v, [d4 + 3])
        x0 = (ps0 + pd0 + g0_v[sl]) * inv_tau
        x1 = (ps1 + pd1 + g1_v[sl]) * inv_tau
        m = jnp.maximum(x0, x1)
        keep = jnp.exp(x0 - m) >= jnp.exp(x1 - m)
        samp_v[sl] = jnp.where(keep, 1.0, 0.0).astype(jnp.float32)
        gidx = ebase + i * 16 + lax.iota(jnp.int32, 16)
        live = keep & (gidx < E)
        dst_v[sl] = jnp.where(live, dv, jnp.int32(200000))
        return carry
    lax.fori_loop(0, NV, _sample, 0)

    pltpu.sync_copy(samp_v, samp_hbm.at[pl.ds(ebase, EPT)])
    plsc.subcore_barrier()

    for b in range(B):
        boff = jnp.int32(b * N)

        def _chunk(ch, carry):
            base = ch * CHUNK

            def _idx(v, carry2):
                vsl = pl.ds(base + v * 16, 16)
                osl = pl.ds(v * 16, 16)
                gi_v[0, osl] = src_v[vsl] + boff
                gi_v[1, osl] = jnp.minimum(dst_v[vsl] + boff,
                                           jnp.int32(TRASH))
                return carry2
            lax.fori_loop(0, CHUNK // 16, _idx, 0)
            pltpu.async_copy(x_hbm.at[gi_v.at[0]], rows_v, sem).wait()
            pltpu.sync_copy(rows_v, agg_sh.at[gi_v.at[1]], add=True)
            return carry
        lax.fori_loop(0, NCH, _chunk, 0)

    plsc.subcore_barrier()
    orows = BN // NS
    pltpu.sync_copy(agg_sh.at[pl.ds(s * orows, orows)],
                    agg_hbm.at[c].at[pl.ds(s * orows, orows)])


_sc_call = functools.partial(
    pl.kernel,
    out_type=(jax.ShapeDtypeStruct((E_PAD,), jnp.float32),
              jax.ShapeDtypeStruct((NC, BN, XPAD), jnp.float32)),
    mesh=plsc.VectorSubcoreMesh(core_axis_name="c", subcore_axis_name="s"),
    scratch_types=[
        pltpu.VMEM((N * 4,), jnp.float32),
        pltpu.VMEM((EPT,), jnp.int32),
        pltpu.VMEM((EPT,), jnp.int32),
        pltpu.VMEM((EPT,), jnp.float32),
        pltpu.VMEM((EPT,), jnp.float32),
        pltpu.VMEM((EPT,), jnp.float32),
        pltpu.VMEM((2, CHUNK), jnp.int32),
        pltpu.VMEM((CHUNK, XPAD), jnp.float32),
        pltpu.VMEM((256, XPAD), jnp.float32),
        pltpu.VMEM_SHARED((AGG_SP, XPAD), jnp.float32),
        pltpu.SemaphoreType.DMA,
    ],
)(_sc_body)


def _readout_body(agg_ref, x_ref, wg_ref, ws_ref, wo_ref, out_ref):
    a = agg_ref[0] + agg_ref[1]
    h = jnp.maximum(
        jnp.dot(a, wg_ref[...], preferred_element_type=jnp.float32)
        + jnp.dot(x_ref[...], ws_ref[...], preferred_element_type=jnp.float32),
        0.0)
    out_ref[...] = jnp.dot(h, wo_ref[...], preferred_element_type=jnp.float32)


def _readout(agg_parts, xpad, wg_pad, ws_pad, wo):
    blk = 4000
    return pl.pallas_call(
        _readout_body,
        grid=(BN // blk,),
        in_specs=[
            pl.BlockSpec((NC, blk, XPAD), lambda i: (0, i, 0)),
            pl.BlockSpec((blk, XPAD), lambda i: (i, 0)),
            pl.BlockSpec((XPAD, HID_FC), lambda i: (0, 0)),
            pl.BlockSpec((XPAD, HID_FC), lambda i: (0, 0)),
            pl.BlockSpec((HID_FC, HORIZON), lambda i: (0, 0)),
        ],
        out_specs=pl.BlockSpec((blk, HORIZON), lambda i: (i, 0)),
        out_shape=jax.ShapeDtypeStruct((BN, HORIZON), jnp.float32),
    )(agg_parts, xpad, wg_pad, ws_pad, wo)


def kernel(inputs, targets, entire_inputs, edge_index, gumbel_noise,
           W1, W2, Wg, Ws, Wo):
    src = edge_index[0].astype(jnp.int32)
    dst = edge_index[1].astype(jnp.int32)
    srcp = jnp.concatenate([src, jnp.zeros((E_PAD - E,), jnp.int32)])
    dstp = jnp.concatenate([dst, jnp.zeros((E_PAD - E,), jnp.int32)])
    gum_t = jnp.concatenate(
        [gumbel_noise, jnp.full((E_PAD - E, 2), 0.5, jnp.float32)]).T
    w2cat = jnp.concatenate([W2[:HID_GL], W2[HID_GL:]], axis=1)

    p, g = _embed(entire_inputs, W1, w2cat, gum_t)

    xpad = jnp.pad(inputs, ((0, 0), (0, XPAD - SEQ)))
    samp_pad, agg_parts = _sc_call(
        p.reshape(-1), srcp, dstp, g[0], g[1], xpad)
    edge_sample = samp_pad[:E]

    wg_pad = jnp.pad(Wg, ((0, XPAD - SEQ), (0, 0)))
    ws_pad = jnp.pad(Ws, ((0, XPAD - SEQ), (0, 0)))
    outputs = _readout(agg_parts, xpad, wg_pad, ws_pad, Wo)
    return (edge_sample, outputs)


# speedup vs baseline: 11.4957x; 11.4957x over previous
import functools

import jax
import jax.numpy as jnp
from jax import lax
from jax.experimental import pallas as pl
from jax.experimental.pallas import tpu as pltpu
from jax.experimental.pallas import tpu_sc as plsc

N = 10000
E = 160000
SEQ = 12
B = 4
BN = B * N
TAU = 0.5
HID_GL = 256
HID_FC = 64
HORIZON = 12

NC, NS = 2, 16
NW = NC * NS
E_PAD = 163840
EPT = E_PAD // NW
NV = EPT // 16
CHUNK = 128
NCH = EPT // CHUNK
AGG_SP = 40960
TRASH = BN
XPAD = 16


def _embed_body(ei_ref, w1_ref, w2c_ref, gum_ref, p_ref, g_ref):
    z = jnp.maximum(
        jnp.dot(ei_ref[...], w1_ref[...], preferred_element_type=jnp.float32),
        0.0)
    p_ref[...] = jnp.dot(z, w2c_ref[...], preferred_element_type=jnp.float32)
    u = gum_ref[...]
    g_ref[...] = -jnp.log(-jnp.log(u + 1e-10) + 1e-10)


def _embed(entire_inputs, w1, w2cat, gum_t):
    blk_n = 1000
    blk_e = E_PAD // 10
    return pl.pallas_call(
        _embed_body,
        grid=(10,),
        in_specs=[
            pl.BlockSpec((blk_n, 1000), lambda i: (i, 0)),
            pl.BlockSpec((1000, HID_GL), lambda i: (0, 0)),
            pl.BlockSpec((HID_GL, 4), lambda i: (0, 0)),
            pl.BlockSpec((2, blk_e), lambda i: (0, i)),
        ],
        out_specs=[
            pl.BlockSpec((blk_n, 4), lambda i: (i, 0)),
            pl.BlockSpec((2, blk_e), lambda i: (0, i)),
        ],
        out_shape=[
            jax.ShapeDtypeStruct((N, 4), jnp.float32),
            jax.ShapeDtypeStruct((2, E_PAD), jnp.float32),
        ],
    )(entire_inputs, w1, w2cat, gum_t)


def _sc_body(p_hbm, src_hbm, dst_hbm, g0_hbm, g1_hbm, x_hbm,
             samp_hbm, agg_hbm,
             p_v, src_v, dst_v, g0_v, g1_v, samp_v,
             gi_v, rows_v, zero_v,
             agg_sh, sem):
    c = lax.axis_index("c")
    s = lax.axis_index("s")
    tile = c * NS + s
    ebase = tile * EPT

    pltpu.sync_copy(src_hbm.at[pl.ds(ebase, EPT)], src_v)
    pltpu.sync_copy(dst_hbm.at[pl.ds(ebase, EPT)], dst_v)
    pltpu.sync_copy(g0_hbm.at[pl.ds(ebase, EPT)], g0_v)
    pltpu.sync_copy(g1_hbm.at[pl.ds(ebase, EPT)], g1_v)
    pltpu.sync_copy(p_hbm, p_v)

    def _zinit(i, carry):
        zero_v[i] = jnp.zeros((16,), jnp.float32)
        return carry
    lax.fori_loop(0, 256, _zinit, 0)
    arows = AGG_SP // NS

    def _zcopy(j, carry):
        pltpu.sync_copy(zero_v, agg_sh.at[pl.ds(s * arows + j * 256, 256)])
        return carry
    lax.fori_loop(0, arows // 256, _zcopy, 0)

    inv_tau = 1.0 / TAU

    def _sample(i, carry):
        sl = pl.ds(i * 16, 16)
        sv = src_v[sl]
        dv = dst_v[sl]
        s4 = sv * 4
        d4 = dv * 4
        ps0 = plsc.load_gather(p_v, [s4])
        ps1 = plsc.load_gather(p_v, [s4 + 1])
        pd0 = plsc.load_gather(p_v, [d4 + 2])
        pd1 = plsc.load_gather(p_v, [d4 + 3])
        x0 = (ps0 + pd0 + g0_v[sl]) * inv_tau
        x1 = (ps1 + pd1 + g1_v[sl]) * inv_tau
        m = jnp.maximum(x0, x1)
        keep = jnp.exp(x0 - m) >= jnp.exp(x1 - m)
        samp_v[sl] = jnp.where(keep, 1.0, 0.0).astype(jnp.float32)
        gidx = ebase + i * 16 + lax.iota(jnp.int32, 16)
        live = keep & (gidx < E)
        dst_v[sl] = jnp.where(live, dv, jnp.int32(200000))
        return carry
    lax.fori_loop(0, NV, _sample, 0)

    pltpu.sync_copy(samp_v, samp_hbm.at[pl.ds(ebase, EPT)])
    plsc.subcore_barrier()

    for b in range(B):
        boff = jnp.int32(b * N)

        def _chunk(ch, carry):
            base = ch * CHUNK

            def _idx(v, carry2):
                vsl = pl.ds(base + v * 16, 16)
                osl = pl.ds(v * 16, 16)
                gi_v[0, osl] = src_v[vsl] + boff
                gi_v[1, osl] = jnp.minimum(dst_v[vsl] + boff,
                                           jnp.int32(TRASH))
                return carry2
            lax.fori_loop(0, CHUNK // 16, _idx, 0)
            pltpu.async_copy(x_hbm.at[gi_v.at[0]], rows_v, sem).wait()
            pltpu.sync_copy(rows_v, agg_sh.at[gi_v.at[1]], add=True)
            return carry
        lax.fori_loop(0, NCH, _chunk, 0)

    plsc.subcore_barrier()
    orows = 2496
    pltpu.sync_copy(agg_sh.at[pl.ds(s * orows, orows)],
                    agg_hbm.at[c].at[pl.ds(s * orows, orows)])

    @pl.when(s == NS - 1)
    def _tail():
        pltpu.sync_copy(agg_sh.at[pl.ds(NS * orows, BN - NS * orows)],
                        agg_hbm.at[c].at[pl.ds(NS * orows, BN - NS * orows)])


_sc_call = functools.partial(
    pl.kernel,
    out_type=(jax.ShapeDtypeStruct((E_PAD,), jnp.float32),
              jax.ShapeDtypeStruct((NC, BN, XPAD), jnp.float32)),
    mesh=plsc.VectorSubcoreMesh(core_axis_name="c", subcore_axis_name="s"),
    compiler_params=pltpu.CompilerParams(needs_layout_passes=False,
                                         use_tc_tiling_on_sc=False),
    scratch_types=[
        pltpu.VMEM((N * 4,), jnp.float32),
        pltpu.VMEM((EPT,), jnp.int32),
        pltpu.VMEM((EPT,), jnp.int32),
        pltpu.VMEM((EPT,), jnp.float32),
        pltpu.VMEM((EPT,), jnp.float32),
        pltpu.VMEM((EPT,), jnp.float32),
        pltpu.VMEM((2, CHUNK), jnp.int32),
        pltpu.VMEM((CHUNK, XPAD), jnp.float32),
        pltpu.VMEM((256, XPAD), jnp.float32),
        pltpu.VMEM_SHARED((AGG_SP, XPAD), jnp.float32),
        pltpu.SemaphoreType.DMA,
    ],
)(_sc_body)


def _readout_body(agg_ref, x_ref, wg_ref, ws_ref, wo_ref, out_ref):
    a = agg_ref[0] + agg_ref[1]
    h = jnp.maximum(
        jnp.dot(a, wg_ref[...], preferred_element_type=jnp.float32)
        + jnp.dot(x_ref[...], ws_ref[...], preferred_element_type=jnp.float32),
        0.0)
    out_ref[...] = jnp.dot(h, wo_ref[...], preferred_element_type=jnp.float32)


def _readout(agg_parts, xpad, wg_pad, ws_pad, wo):
    blk = 4000
    return pl.pallas_call(
        _readout_body,
        grid=(BN // blk,),
        in_specs=[
            pl.BlockSpec((NC, blk, XPAD), lambda i: (0, i, 0)),
            pl.BlockSpec((blk, XPAD), lambda i: (i, 0)),
            pl.BlockSpec((XPAD, HID_FC), lambda i: (0, 0)),
            pl.BlockSpec((XPAD, HID_FC), lambda i: (0, 0)),
            pl.BlockSpec((HID_FC, HORIZON), lambda i: (0, 0)),
        ],
        out_specs=pl.BlockSpec((blk, HORIZON), lambda i: (i, 0)),
        out_shape=jax.ShapeDtypeStruct((BN, HORIZON), jnp.float32),
    )(agg_parts, xpad, wg_pad, ws_pad, wo)


def kernel(inputs, targets, entire_inputs, edge_index, gumbel_noise,
           W1, W2, Wg, Ws, Wo):
    src = edge_index[0].astype(jnp.int32)
    dst = edge_index[1].astype(jnp.int32)
    srcp = jnp.concatenate([src, jnp.zeros((E_PAD - E,), jnp.int32)])
    dstp = jnp.concatenate([dst, jnp.zeros((E_PAD - E,), jnp.int32)])
    gum_t = jnp.concatenate(
        [gumbel_noise, jnp.full((E_PAD - E, 2), 0.5, jnp.float32)]).T
    w2cat = jnp.concatenate([W2[:HID_GL], W2[HID_GL:]], axis=1)

    p, g = _embed(entire_inputs, W1, w2cat, gum_t)

    xpad = jnp.pad(inputs, ((0, 0), (0, XPAD - SEQ)))
    samp_pad, agg_parts = _sc_call(
        p.reshape(-1), srcp, dstp, g[0], g[1], xpad)
    edge_sample = samp_pad[:E]

    wg_pad = jnp.pad(Wg, ((0, XPAD - SEQ), (0, 0)))
    ws_pad = jnp.pad(Ws, ((0, XPAD - SEQ), (0, 0)))
    outputs = _readout(agg_parts, xpad, wg_pad, ws_pad, Wo)
    return (edge_sample, outputs)


# speedup vs baseline: 11.9320x; 1.0380x over previous
import functools

import jax
import jax.numpy as jnp
from jax import lax
from jax.experimental import pallas as pl
from jax.experimental.pallas import tpu as pltpu
from jax.experimental.pallas import tpu_sc as plsc

N = 10000
E = 160000
SEQ = 12
B = 4
BN = B * N
TAU = 0.5
HID_GL = 256
HID_FC = 64
HORIZON = 12

NC, NS = 2, 16
NW = NC * NS
E_PAD = 163840
EPT = E_PAD // NW
NV = EPT // 16
CHUNK = 128
NCH = EPT // CHUNK
K = 8
AGG_SP = 40960
TRASH = BN
XPAD = 16


def _embed_body(ei_ref, w1_ref, w2c_ref, gum_ref, p_ref, g_ref):
    z = jnp.maximum(
        jnp.dot(ei_ref[...], w1_ref[...], preferred_element_type=jnp.float32),
        0.0)
    p_ref[...] = jnp.dot(z, w2c_ref[...], preferred_element_type=jnp.float32)
    u = gum_ref[...]
    g_ref[...] = -jnp.log(-jnp.log(u + 1e-10) + 1e-10)


def _embed(entire_inputs, w1, w2cat, gum_t):
    blk_n = 1000
    blk_e = E_PAD // 10
    return pl.pallas_call(
        _embed_body,
        grid=(10,),
        in_specs=[
            pl.BlockSpec((blk_n, 1000), lambda i: (i, 0)),
            pl.BlockSpec((1000, HID_GL), lambda i: (0, 0)),
            pl.BlockSpec((HID_GL, 4), lambda i: (0, 0)),
            pl.BlockSpec((2, blk_e), lambda i: (0, i)),
        ],
        out_specs=[
            pl.BlockSpec((blk_n, 4), lambda i: (i, 0)),
            pl.BlockSpec((2, blk_e), lambda i: (0, i)),
        ],
        out_shape=[
            jax.ShapeDtypeStruct((N, 4), jnp.float32),
            jax.ShapeDtypeStruct((2, E_PAD), jnp.float32),
        ],
    )(entire_inputs, w1, w2cat, gum_t)


def _sc_body(p_hbm, src_hbm, dst_hbm, g0_hbm, g1_hbm, x_hbm,
             samp_hbm, agg_hbm,
             p_v, src_v, dst_v, g0_v, g1_v, samp_v,
             gsrc_v, gdst_v, rows_v, zero_v,
             agg_sh, sem, sem2):
    c = lax.axis_index("c")
    s = lax.axis_index("s")
    tile = c * NS + s
    ebase = tile * EPT

    pltpu.sync_copy(src_hbm.at[pl.ds(ebase, EPT)], src_v)
    pltpu.sync_copy(dst_hbm.at[pl.ds(ebase, EPT)], dst_v)
    pltpu.sync_copy(g0_hbm.at[pl.ds(ebase, EPT)], g0_v)
    pltpu.sync_copy(g1_hbm.at[pl.ds(ebase, EPT)], g1_v)
    pltpu.sync_copy(p_hbm, p_v)

    def _zinit(i, carry):
        zero_v[i] = jnp.zeros((16,), jnp.float32)
        return carry
    lax.fori_loop(0, 256, _zinit, 0)
    arows = AGG_SP // NS

    def _zcopy(j, carry):
        pltpu.sync_copy(zero_v, agg_sh.at[pl.ds(s * arows + j * 256, 256)])
        return carry
    lax.fori_loop(0, arows // 256, _zcopy, 0)

    inv_tau = 1.0 / TAU

    def _sample(i, carry):
        sl = pl.ds(i * 16, 16)
        sv = src_v[sl]
        dv = dst_v[sl]
        s4 = sv * 4
        d4 = dv * 4
        ps0 = plsc.load_gather(p_v, [s4])
        ps1 = plsc.load_gather(p_v, [s4 + 1])
        pd0 = plsc.load_gather(p_v, [d4 + 2])
        pd1 = plsc.load_gather(p_v, [d4 + 3])
        x0 = (ps0 + pd0 + g0_v[sl]) * inv_tau
        x1 = (ps1 + pd1 + g1_v[sl]) * inv_tau
        m = jnp.maximum(x0, x1)
        keep = jnp.exp(x0 - m) >= jnp.exp(x1 - m)
        samp_v[sl] = jnp.where(keep, 1.0, 0.0).astype(jnp.float32)
        gidx = ebase + i * 16 + lax.iota(jnp.int32, 16)
        live = keep & (gidx < E)
        dst_v[sl] = jnp.where(live, dv, jnp.int32(200000))
        return carry
    lax.fori_loop(0, NV, _sample, 0)

    pltpu.sync_copy(samp_v, samp_hbm.at[pl.ds(ebase, EPT)])
    plsc.subcore_barrier()

    for b in range(B):
        boff = jnp.int32(b * N)

        def _sup(sc_i, carry):
            sbase = sc_i * (K * CHUNK)
            gets = []
            for k in range(K):
                def _idx(v, carry2, k=k):
                    vsl = pl.ds(sbase + k * CHUNK + v * 16, 16)
                    osl = pl.ds(v * 16, 16)
                    gsrc_v[k, osl] = src_v[vsl] + boff
                    gdst_v[k, osl] = jnp.minimum(dst_v[vsl] + boff,
                                                 jnp.int32(TRASH))
                    return carry2
                lax.fori_loop(0, CHUNK // 16, _idx, 0)
                gets.append(pltpu.async_copy(
                    x_hbm.at[gsrc_v.at[k]], rows_v.at[k], sem))
            puts = []
            for k in range(K):
                gets[k].wait()
                puts.append(pltpu.async_copy(
                    rows_v.at[k], agg_sh.at[gdst_v.at[k]], sem2, add=True))
            for cp in puts:
                cp.wait()
            return carry
        lax.fori_loop(0, NCH // K, _sup, 0)

    plsc.subcore_barrier()
    orows = 2496
    pltpu.sync_copy(agg_sh.at[pl.ds(s * orows, orows)],
                    agg_hbm.at[c].at[pl.ds(s * orows, orows)])

    @pl.when(s == NS - 1)
    def _tail():
        pltpu.sync_copy(agg_sh.at[pl.ds(NS * orows, BN - NS * orows)],
                        agg_hbm.at[c].at[pl.ds(NS * orows, BN - NS * orows)])


_sc_call = functools.partial(
    pl.kernel,
    out_type=(jax.ShapeDtypeStruct((E_PAD,), jnp.float32),
              jax.ShapeDtypeStruct((NC, BN, XPAD), jnp.float32)),
    mesh=plsc.VectorSubcoreMesh(core_axis_name="c", subcore_axis_name="s"),
    compiler_params=pltpu.CompilerParams(needs_layout_passes=False,
                                         use_tc_tiling_on_sc=False),
    scratch_types=[
        pltpu.VMEM((N * 4,), jnp.float32),
        pltpu.VMEM((EPT,), jnp.int32),
        pltpu.VMEM((EPT,), jnp.int32),
        pltpu.VMEM((EPT,), jnp.float32),
        pltpu.VMEM((EPT,), jnp.float32),
        pltpu.VMEM((EPT,), jnp.float32),
        pltpu.VMEM((K, CHUNK), jnp.int32),
        pltpu.VMEM((K, CHUNK), jnp.int32),
        pltpu.VMEM((K, CHUNK, XPAD), jnp.float32),
        pltpu.VMEM((256, XPAD), jnp.float32),
        pltpu.VMEM_SHARED((AGG_SP, XPAD), jnp.float32),
        pltpu.SemaphoreType.DMA,
        pltpu.SemaphoreType.DMA,
    ],
)(_sc_body)


def _readout_body(agg_ref, x_ref, wg_ref, ws_ref, wo_ref, out_ref):
    a = agg_ref[0] + agg_ref[1]
    h = jnp.maximum(
        jnp.dot(a, wg_ref[...], preferred_element_type=jnp.float32)
        + jnp.dot(x_ref[...], ws_ref[...], preferred_element_type=jnp.float32),
        0.0)
    out_ref[...] = jnp.dot(h, wo_ref[...], preferred_element_type=jnp.float32)


def _readout(agg_parts, xpad, wg_pad, ws_pad, wo):
    blk = 4000
    return pl.pallas_call(
        _readout_body,
        grid=(BN // blk,),
        in_specs=[
            pl.BlockSpec((NC, blk, XPAD), lambda i: (0, i, 0)),
            pl.BlockSpec((blk, XPAD), lambda i: (i, 0)),
            pl.BlockSpec((XPAD, HID_FC), lambda i: (0, 0)),
            pl.BlockSpec((XPAD, HID_FC), lambda i: (0, 0)),
            pl.BlockSpec((HID_FC, HORIZON), lambda i: (0, 0)),
        ],
        out_specs=pl.BlockSpec((blk, HORIZON), lambda i: (i, 0)),
        out_shape=jax.ShapeDtypeStruct((BN, HORIZON), jnp.float32),
    )(agg_parts, xpad, wg_pad, ws_pad, wo)


def kernel(inputs, targets, entire_inputs, edge_index, gumbel_noise,
           W1, W2, Wg, Ws, Wo):
    src = edge_index[0].astype(jnp.int32)
    dst = edge_index[1].astype(jnp.int32)
    srcp = jnp.concatenate([src, jnp.zeros((E_PAD - E,), jnp.int32)])
    dstp = jnp.concatenate([dst, jnp.zeros((E_PAD - E,), jnp.int32)])
    gum_t = jnp.concatenate(
        [gumbel_noise, jnp.full((E_PAD - E, 2), 0.5, jnp.float32)]).T
    w2cat = jnp.concatenate([W2[:HID_GL], W2[HID_GL:]], axis=1)

    p, g = _embed(entire_inputs, W1, w2cat, gum_t)

    xpad = jnp.pad(inputs, ((0, 0), (0, XPAD - SEQ)))
    samp_pad, agg_parts = _sc_call(
        p.reshape(-1), srcp, dstp, g[0], g[1], xpad)
    edge_sample = samp_pad[:E]

    wg_pad = jnp.pad(Wg, ((0, XPAD - SEQ), (0, 0)))
    ws_pad = jnp.pad(Ws, ((0, XPAD - SEQ), (0, 0)))
    outputs = _readout(agg_parts, xpad, wg_pad, ws_pad, Wo)
    return (edge_sample, outputs)


# speedup vs baseline: 11.9598x; 1.0023x over previous
import functools

import jax
import jax.numpy as jnp
from jax import lax
from jax.experimental import pallas as pl
from jax.experimental.pallas import tpu as pltpu
from jax.experimental.pallas import tpu_sc as plsc

N = 10000
E = 160000
SEQ = 12
B = 4
BN = B * N
TAU = 0.5
HID_GL = 256
HID_FC = 64
HORIZON = 12

NC, NS = 2, 16
NW = NC * NS
E_PAD = 163840
EPT = E_PAD // NW
NV = EPT // 16
CHUNK = 128
NCH = EPT // CHUNK
K = 8
AGG_SP = 40960
TRASH = BN
XPAD = 16


def _embed_body(ei_ref, w1_ref, w2c_ref, gum_ref, p_ref, g_ref):
    z = jnp.maximum(
        jnp.dot(ei_ref[...], w1_ref[...], preferred_element_type=jnp.float32),
        0.0)
    p_ref[...] = jnp.dot(z, w2c_ref[...], preferred_element_type=jnp.float32)
    u = gum_ref[...]
    g_ref[...] = -jnp.log(-jnp.log(u + 1e-10) + 1e-10)


def _embed(entire_inputs, w1, w2cat, gum_t):
    blk_n = 1000
    blk_e = E_PAD // 10
    return pl.pallas_call(
        _embed_body,
        grid=(10,),
        in_specs=[
            pl.BlockSpec((blk_n, 1000), lambda i: (i, 0)),
            pl.BlockSpec((1000, HID_GL), lambda i: (0, 0)),
            pl.BlockSpec((HID_GL, 4), lambda i: (0, 0)),
            pl.BlockSpec((2, blk_e), lambda i: (0, i)),
        ],
        out_specs=[
            pl.BlockSpec((blk_n, 4), lambda i: (i, 0)),
            pl.BlockSpec((2, blk_e), lambda i: (0, i)),
        ],
        out_shape=[
            jax.ShapeDtypeStruct((N, 4), jnp.float32),
            jax.ShapeDtypeStruct((2, E_PAD), jnp.float32),
        ],
    )(entire_inputs, w1, w2cat, gum_t)


def _sc_body(p_hbm, src_hbm, dst_hbm, g0_hbm, g1_hbm, x_hbm,
             samp_hbm, agg_hbm,
             p_v, src_v, dst_v, g0_v, g1_v, samp_v,
             gsrc_v, gdst_v, rows_v, zero_v,
             agg_sh, sem, sem2):
    c = lax.axis_index("c")
    s = lax.axis_index("s")
    tile = c * NS + s
    ebase = tile * EPT

    with jax.named_scope("sc_stage"):
        pltpu.sync_copy(src_hbm.at[pl.ds(ebase, EPT)], src_v)
        pltpu.sync_copy(dst_hbm.at[pl.ds(ebase, EPT)], dst_v)
        pltpu.sync_copy(g0_hbm.at[pl.ds(ebase, EPT)], g0_v)
        pltpu.sync_copy(g1_hbm.at[pl.ds(ebase, EPT)], g1_v)
        pltpu.sync_copy(p_hbm, p_v)

    with jax.named_scope("sc_zero"):
        def _zinit(i, carry):
            zero_v[i] = jnp.zeros((16,), jnp.float32)
            return carry
        lax.fori_loop(0, 256, _zinit, 0)
        arows = AGG_SP // NS

        def _zcopy(j, carry):
            pltpu.sync_copy(zero_v,
                            agg_sh.at[pl.ds(s * arows + j * 256, 256)])
            return carry
        lax.fori_loop(0, arows // 256, _zcopy, 0)

    inv_tau = 1.0 / TAU

    def _sample(i, carry):
        sl = pl.ds(i * 16, 16)
        sv = src_v[sl]
        dv = dst_v[sl]
        s4 = sv * 4
        d4 = dv * 4
        ps0 = plsc.load_gather(p_v, [s4])
        ps1 = plsc.load_gather(p_v, [s4 + 1])
        pd0 = plsc.load_gather(p_v, [d4 + 2])
        pd1 = plsc.load_gather(p_v, [d4 + 3])
        x0 = (ps0 + pd0 + g0_v[sl]) * inv_tau
        x1 = (ps1 + pd1 + g1_v[sl]) * inv_tau
        m = jnp.maximum(x0, x1)
        keep = jnp.exp(x0 - m) >= jnp.exp(x1 - m)
        samp_v[sl] = jnp.where(keep, 1.0, 0.0).astype(jnp.float32)
        gidx = ebase + i * 16 + lax.iota(jnp.int32, 16)
        live = keep & (gidx < E)
        dst_v[sl] = jnp.where(live, dv, jnp.int32(200000))
        return carry
    with jax.named_scope("sc_sample"):
        lax.fori_loop(0, NV, _sample, 0)
        pltpu.sync_copy(samp_v, samp_hbm.at[pl.ds(ebase, EPT)])
    plsc.subcore_barrier()

    for b in range(B):
        boff = jnp.int32(b * N)

        def _sup(sc_i, carry, b=b):
            sbase = sc_i * (K * CHUNK)
            gets = []
            for k in range(K):
                def _idx(v, carry2, k=k):
                    vsl = pl.ds(sbase + k * CHUNK + v * 16, 16)
                    osl = pl.ds(v * 16, 16)
                    gsrc_v[k, osl] = src_v[vsl] + boff
                    gdst_v[k, osl] = jnp.minimum(dst_v[vsl] + boff,
                                                 jnp.int32(TRASH))
                    return carry2
                lax.fori_loop(0, CHUNK // 16, _idx, 0)
                gets.append(pltpu.async_copy(
                    x_hbm.at[gsrc_v.at[k]], rows_v.at[k], sem))
            puts = []
            for k in range(K):
                gets[k].wait()
                puts.append(pltpu.async_copy(
                    rows_v.at[k], agg_sh.at[gdst_v.at[k]], sem2, add=True))
            for cp in puts:
                cp.wait()
            return carry
        with jax.named_scope(f"sc_scatter{b}"):
            lax.fori_loop(0, NCH // K, _sup, 0)

    plsc.subcore_barrier()
    with jax.named_scope("sc_writeout"):
        orows = 2496
        pltpu.sync_copy(agg_sh.at[pl.ds(s * orows, orows)],
                        agg_hbm.at[c].at[pl.ds(s * orows, orows)])

        @pl.when(s == NS - 1)
        def _tail():
            pltpu.sync_copy(
                agg_sh.at[pl.ds(NS * orows, BN - NS * orows)],
                agg_hbm.at[c].at[pl.ds(NS * orows, BN - NS * orows)])


_sc_call = functools.partial(
    pl.kernel,
    out_type=(jax.ShapeDtypeStruct((E_PAD,), jnp.float32),
              jax.ShapeDtypeStruct((NC, BN, XPAD), jnp.float32)),
    mesh=plsc.VectorSubcoreMesh(core_axis_name="c", subcore_axis_name="s"),
    compiler_params=pltpu.CompilerParams(needs_layout_passes=False,
                                         use_tc_tiling_on_sc=False),
    scratch_types=[
        pltpu.VMEM((N * 4,), jnp.float32),
        pltpu.VMEM((EPT,), jnp.int32),
        pltpu.VMEM((EPT,), jnp.int32),
        pltpu.VMEM((EPT,), jnp.float32),
        pltpu.VMEM((EPT,), jnp.float32),
        pltpu.VMEM((EPT,), jnp.float32),
        pltpu.VMEM((K, CHUNK), jnp.int32),
        pltpu.VMEM((K, CHUNK), jnp.int32),
        pltpu.VMEM((K, CHUNK, XPAD), jnp.float32),
        pltpu.VMEM((256, XPAD), jnp.float32),
        pltpu.VMEM_SHARED((AGG_SP, XPAD), jnp.float32),
        pltpu.SemaphoreType.DMA,
        pltpu.SemaphoreType.DMA,
    ],
)(_sc_body)


def _readout_body(agg_ref, x_ref, wg_ref, ws_ref, wo_ref, out_ref):
    a = agg_ref[0] + agg_ref[1]
    h = jnp.maximum(
        jnp.dot(a, wg_ref[...], preferred_element_type=jnp.float32)
        + jnp.dot(x_ref[...], ws_ref[...], preferred_element_type=jnp.float32),
        0.0)
    out_ref[...] = jnp.dot(h, wo_ref[...], preferred_element_type=jnp.float32)


def _readout(agg_parts, xpad, wg_pad, ws_pad, wo):
    blk = 4000
    return pl.pallas_call(
        _readout_body,
        grid=(BN // blk,),
        in_specs=[
            pl.BlockSpec((NC, blk, XPAD), lambda i: (0, i, 0)),
            pl.BlockSpec((blk, XPAD), lambda i: (i, 0)),
            pl.BlockSpec((XPAD, HID_FC), lambda i: (0, 0)),
            pl.BlockSpec((XPAD, HID_FC), lambda i: (0, 0)),
            pl.BlockSpec((HID_FC, HORIZON), lambda i: (0, 0)),
        ],
        out_specs=pl.BlockSpec((blk, HORIZON), lambda i: (i, 0)),
        out_shape=jax.ShapeDtypeStruct((BN, HORIZON), jnp.float32),
    )(agg_parts, xpad, wg_pad, ws_pad, wo)


def kernel(inputs, targets, entire_inputs, edge_index, gumbel_noise,
           W1, W2, Wg, Ws, Wo):
    src = edge_index[0].astype(jnp.int32)
    dst = edge_index[1].astype(jnp.int32)
    srcp = jnp.concatenate([src, jnp.zeros((E_PAD - E,), jnp.int32)])
    dstp = jnp.concatenate([dst, jnp.zeros((E_PAD - E,), jnp.int32)])
    gum_t = jnp.concatenate(
        [gumbel_noise, jnp.full((E_PAD - E, 2), 0.5, jnp.float32)]).T
    w2cat = jnp.concatenate([W2[:HID_GL], W2[HID_GL:]], axis=1)

    p, g = _embed(entire_inputs, W1, w2cat, gum_t)

    xpad = jnp.pad(inputs, ((0, 0), (0, XPAD - SEQ)))
    samp_pad, agg_parts = _sc_call(
        p.reshape(-1), srcp, dstp, g[0], g[1], xpad)
    edge_sample = samp_pad[:E]

    wg_pad = jnp.pad(Wg, ((0, XPAD - SEQ), (0, 0)))
    ws_pad = jnp.pad(Ws, ((0, XPAD - SEQ), (0, 0)))
    outputs = _readout(agg_parts, xpad, wg_pad, ws_pad, Wo)
    return (edge_sample, outputs)


# speedup vs baseline: 12.8088x; 1.0710x over previous
import functools

import jax
import jax.numpy as jnp
from jax import lax
from jax.experimental import pallas as pl
from jax.experimental.pallas import tpu as pltpu
from jax.experimental.pallas import tpu_sc as plsc

N = 10000
E = 160000
SEQ = 12
B = 4
BN = B * N
TAU = 0.5
HID_GL = 256
HID_FC = 64
HORIZON = 12

NC, NS = 2, 16
NW = NC * NS
EPT = 5120
NV = EPT // 16
CHUNK = 128
K = 2
AGG_ROWS = 10240
XPAD = 16


def _embed_body(ei_ref, w1_ref, w2c_ref, gum_ref, x_ref,
                p_ref, g_ref, xb_ref):
    z = jnp.maximum(
        jnp.dot(ei_ref[...], w1_ref[...], preferred_element_type=jnp.float32),
        0.0)
    p_ref[...] = jnp.dot(z, w2c_ref[...], preferred_element_type=jnp.float32)
    u = gum_ref[...]
    g_ref[...] = -jnp.log(-jnp.log(u + 1e-10) + 1e-10)
    z4 = jnp.zeros((x_ref.shape[1], XPAD - SEQ), jnp.float32)
    parts = []
    for b in range(B):
        parts.append(x_ref[b])
        parts.append(z4)
    xb_ref[...] = jnp.concatenate(parts, axis=-1)


def _embed(entire_inputs, w1, w2cat, gum, x3):
    return pl.pallas_call(
        _embed_body,
        grid=(10,),
        in_specs=[
            pl.BlockSpec((N // 10, 1000), lambda i: (i, 0)),
            pl.BlockSpec((1000, HID_GL), lambda i: (0, 0)),
            pl.BlockSpec((HID_GL, 4), lambda i: (0, 0)),
            pl.BlockSpec((E // 10, 2), lambda i: (i, 0)),
            pl.BlockSpec((B, N // 10, SEQ), lambda i: (0, i, 0)),
        ],
        out_specs=[
            pl.BlockSpec((N // 10, 4), lambda i: (i, 0)),
            pl.BlockSpec((E // 10, 2), lambda i: (i, 0)),
            pl.BlockSpec((N // 10, B * XPAD), lambda i: (i, 0)),
        ],
        out_shape=[
            jax.ShapeDtypeStruct((N, 4), jnp.float32),
            jax.ShapeDtypeStruct((E, 2), jnp.float32),
            jax.ShapeDtypeStruct((N, B * XPAD), jnp.float32),
        ],
    )(entire_inputs, w1, w2cat, gum, x3)


def _sc_body(p_hbm, src_hbm, dst_hbm, g2_hbm, x_hbm,
             samp_hbm, agg_hbm,
             p_v, src_v, dst_v, g_v, samp_v,
             sidx_v, rows_v,
             agg_sh, sem, sem2):
    src_c = src_v
    dst_c = dst_v
    c = lax.axis_index("c")
    s = lax.axis_index("s")
    tile = c * NS + s
    own_base = tile * EPT
    sbase = jnp.minimum(own_base, E - EPT)

    with jax.named_scope("sc_stage"):
        pltpu.sync_copy(src_hbm.at[pl.ds(sbase, EPT)], src_v)
        pltpu.sync_copy(dst_hbm.at[pl.ds(sbase, EPT)], dst_v)
        pltpu.sync_copy(g2_hbm.at[pl.ds(2 * sbase, 2 * EPT)], g_v)
        pltpu.sync_copy(p_hbm, p_v)

    with jax.named_scope("sc_zero"):
        def _zinit(i, carry):
            for b in range(B):
                rows_v[0, i, pl.ds(b * XPAD, 16)] = jnp.zeros(
                    (16,), jnp.float32)
            return carry
        lax.fori_loop(0, CHUNK, _zinit, 0)

        def _zcopy(j, carry):
            pltpu.sync_copy(
                rows_v.at[0],
                agg_sh.at[pl.ds(s * (AGG_ROWS // NS) + j * CHUNK, CHUNK)])
            return carry
        lax.fori_loop(0, AGG_ROWS // NS // CHUNK, _zcopy, 0)

    inv_tau = 1.0 / TAU
    trash = jnp.int32(N) + s

    def _sample(i, off):
        sl = pl.ds(i * 16, 16)
        sv = src_v[sl]
        dv = dst_v[sl]
        eidx = i * 16 + lax.iota(jnp.int32, 16)
        g0 = plsc.load_gather(g_v, [2 * eidx])
        g1 = plsc.load_gather(g_v, [2 * eidx + 1])
        s4 = sv * 4
        d4 = dv * 4
        ps0 = plsc.load_gather(p_v, [s4])
        ps1 = plsc.load_gather(p_v, [s4 + 1])
        pd0 = plsc.load_gather(p_v, [d4 + 2])
        pd1 = plsc.load_gather(p_v, [d4 + 3])
        x0 = (ps0 + pd0 + g0) * inv_tau
        x1 = (ps1 + pd1 + g1) * inv_tau
        m = jnp.maximum(x0, x1)
        keep = jnp.exp(x0 - m) >= jnp.exp(x1 - m)
        samp_v[sl] = jnp.where(keep, 1.0, 0.0).astype(jnp.float32)
        live = keep & (sbase + eidx >= own_base)
        src_c[sl] = jnp.zeros((16,), jnp.int32)
        dst_c[sl] = jnp.full((16,), 200000, jnp.int32)
        cnt = jnp.max(plsc.all_reduce_population_count(live))
        plsc.store_compressed(src_c.at[pl.ds(off, 16)], sv, mask=live)
        plsc.store_compressed(dst_c.at[pl.ds(off, 16)], dv, mask=live)
        return off + cnt

    with jax.named_scope("sc_sample"):
        nlive = lax.fori_loop(0, NV, _sample, jnp.int32(0))
        pltpu.sync_copy(samp_v, samp_hbm.at[pl.ds(sbase, EPT)])
    plsc.subcore_barrier()

    def _sup(sc_i, carry):
        base = sc_i * (K * CHUNK)
        gets = []
        for k in range(K):
            def _mkidx(v, carry2, k=k):
                vsl = pl.ds(base + k * CHUNK + v * 16, 16)
                sidx_v[k, pl.ds(v * 16, 16)] = jnp.minimum(dst_c[vsl], trash)
                return carry2
            lax.fori_loop(0, CHUNK // 16, _mkidx, 0)
            gets.append(pltpu.async_copy(
                x_hbm.at[src_c.at[pl.ds(base + k * CHUNK, CHUNK)]],
                rows_v.at[k], sem))
        puts = []
        for k in range(K):
            gets[k].wait()
            puts.append(pltpu.async_copy(
                rows_v.at[k], agg_sh.at[sidx_v.at[k]], sem2, add=True))
        for cp in puts:
            cp.wait()
        return carry

    with jax.named_scope("sc_scatter"):
        nsup = (nlive + (K * CHUNK - 1)) // (K * CHUNK)
        lax.fori_loop(0, nsup, _sup, 0)

    plsc.subcore_barrier()
    with jax.named_scope("sc_writeout"):
        nrow = 640

        @pl.when(s < NS - 1)
        def _out_main():
            pltpu.sync_copy(agg_sh.at[pl.ds(s * nrow, nrow)],
                            agg_hbm.at[c].at[pl.ds(s * nrow, nrow)])

        @pl.when(s == NS - 1)
        def _out_tail():
            pltpu.sync_copy(
                agg_sh.at[pl.ds((NS - 1) * nrow, N - (NS - 1) * nrow)],
                agg_hbm.at[c].at[pl.ds((NS - 1) * nrow, N - (NS - 1) * nrow)])


_sc_call = functools.partial(
    pl.kernel,
    out_type=(jax.ShapeDtypeStruct((E,), jnp.float32),
              jax.ShapeDtypeStruct((NC, N, B * XPAD), jnp.float32)),
    mesh=plsc.VectorSubcoreMesh(core_axis_name="c", subcore_axis_name="s"),
    compiler_params=pltpu.CompilerParams(needs_layout_passes=False,
                                         use_tc_tiling_on_sc=False),
    scratch_types=[
        pltpu.VMEM((N * 4,), jnp.float32),
        pltpu.VMEM((EPT,), jnp.int32),
        pltpu.VMEM((EPT,), jnp.int32),
        pltpu.VMEM((2 * EPT,), jnp.float32),
        pltpu.VMEM((EPT,), jnp.float32),
        pltpu.VMEM((K, CHUNK), jnp.int32),
        pltpu.VMEM((K, CHUNK, B * XPAD), jnp.float32),
        pltpu.VMEM_SHARED((AGG_ROWS, B * XPAD), jnp.float32),
        pltpu.SemaphoreType.DMA,
        pltpu.SemaphoreType.DMA,
    ],
)(_sc_body)


def _readout_body(agg_ref, x_ref, wg_ref, ws_ref, wo_ref, out_ref):
    a = agg_ref[0] + agg_ref[1]
    for b in range(B):
        ab = a[:, b * XPAD:(b + 1) * XPAD]
        h = jnp.maximum(
            jnp.dot(ab, wg_ref[...], preferred_element_type=jnp.float32)
            + jnp.dot(x_ref[b], ws_ref[...],
                      preferred_element_type=jnp.float32),
            0.0)
        out_ref[b] = jnp.dot(h, wo_ref[...],
                             preferred_element_type=jnp.float32)


def _readout(agg_parts, x, wg_pad, ws, wo):
    blk = 1000
    return pl.pallas_call(
        _readout_body,
        grid=(N // blk,),
        in_specs=[
            pl.BlockSpec((NC, blk, B * XPAD), lambda i: (0, i, 0)),
            pl.BlockSpec((B, blk, SEQ), lambda i: (0, i, 0)),
            pl.BlockSpec((XPAD, HID_FC), lambda i: (0, 0)),
            pl.BlockSpec((SEQ, HID_FC), lambda i: (0, 0)),
            pl.BlockSpec((HID_FC, HORIZON), lambda i: (0, 0)),
        ],
        out_specs=pl.BlockSpec((B, blk, HORIZON), lambda i: (0, i, 0)),
        out_shape=jax.ShapeDtypeStruct((B, N, HORIZON), jnp.float32),
    )(agg_parts, x, wg_pad, ws, wo)


def kernel(inputs, targets, entire_inputs, edge_index, gumbel_noise,
           W1, W2, Wg, Ws, Wo):
    src = edge_index[0].astype(jnp.int32)
    dst = edge_index[1].astype(jnp.int32)
    w2cat = jnp.concatenate([W2[:HID_GL], W2[HID_GL:]], axis=1)

    x3 = inputs.reshape(B, N, SEQ)
    p, g, xb = _embed(entire_inputs, W1, w2cat, gumbel_noise, x3)

    edge_sample, agg_parts = _sc_call(
        p.reshape(-1), src, dst, g.reshape(-1), xb)

    wg_pad = jnp.pad(Wg, ((0, XPAD - SEQ), (0, 0)))
    outputs = _readout(agg_parts, x3, wg_pad, Ws, Wo)
    return (edge_sample, outputs.reshape(BN, HORIZON))
